# Initial kernel scaffold; baseline (speedup 1.0000x reference)
import jax
import jax.numpy as jnp
from jax.experimental import pallas as pl


def kernel(x, token_to_expert_indices, weights, Wg, Wu, Wd):
    raise NotImplementedError("write your pallas kernel here")


# speedup vs baseline: 1.9713x; 1.9713x over previous
import jax
import jax.numpy as jnp
from jax.experimental import pallas as pl
from jax.experimental.pallas import tpu as pltpu


def _dense_body(idx_ref, w_ref, x_ref, wg_ref, wu_ref, wd_ref, y_ref):
    e = pl.program_id(0)

    @pl.when(e == 0)
    def _init():
        y_ref[...] = jnp.zeros_like(y_ref)

    x = x_ref[...]
    g = jnp.dot(x, wg_ref[0], preferred_element_type=jnp.float32)
    u = jnp.dot(x, wu_ref[0], preferred_element_type=jnp.float32)
    h = (g * jax.nn.sigmoid(g)) * u
    o = jnp.dot(h, wd_ref[0], preferred_element_type=jnp.float32)
    idx = idx_ref[...]
    w = w_ref[...]
    w_e = jnp.sum(jnp.where(idx == e, w, 0.0), axis=1, keepdims=True)
    y_ref[...] += o * w_e


def kernel(x, token_to_expert_indices, weights, Wg, Wu, Wd):
    T, D = x.shape
    E, _, H = Wg.shape
    idx = token_to_expert_indices.astype(jnp.int32)
    grid = (E,)
    return pl.pallas_call(
        _dense_body,
        grid=grid,
        in_specs=[
            pl.BlockSpec((T, idx.shape[1]), lambda e: (0, 0)),
            pl.BlockSpec((T, weights.shape[1]), lambda e: (0, 0)),
            pl.BlockSpec((T, D), lambda e: (0, 0)),
            pl.BlockSpec((1, D, H), lambda e: (e, 0, 0)),
            pl.BlockSpec((1, D, H), lambda e: (e, 0, 0)),
            pl.BlockSpec((1, H, D), lambda e: (e, 0, 0)),
        ],
        out_specs=pl.BlockSpec((T, D), lambda e: (0, 0)),
        out_shape=jax.ShapeDtypeStruct((T, D), jnp.float32),
    )(idx, weights, x, Wg, Wu, Wd)
